# Initial kernel scaffold; baseline (speedup 1.0000x reference)
import jax
import jax.numpy as jnp
from jax.experimental import pallas as pl


def kernel(x, edge_index, Wk, bk, Wq, bq, Wv, bv, Wff, bff):
    raise NotImplementedError("write your pallas kernel here")


# speedup vs baseline: 8.9489x; 8.9489x over previous
import jax
import jax.numpy as jnp
from jax import lax
from jax.experimental import pallas as pl
from jax.experimental.pallas import tpu as pltpu
from jax.experimental.pallas import tpu_sc as plsc

N = 10000
E = 320000
D = 128
H = 8
DK = 64
DV = 64
NC = 2
NS = 16
PP = H // NC
B = 80
ET = E // NS
NCH = ET // B
NR = N // NS
ZB = 125
SLAB_W = 80
BN = 1000


def _proj_body(x_ref, wq_ref, bq_ref, wkv_ref, bkv_ref, q_ref, kv_ref):
    xb = x_ref[...]
    q_ref[...] = jnp.dot(xb, wq_ref[...], precision=lax.Precision.HIGHEST,
                         preferred_element_type=jnp.float32) + bq_ref[...]
    kv_ref[...] = jnp.dot(xb, wkv_ref[...], precision=lax.Precision.HIGHEST,
                          preferred_element_type=jnp.float32) + bkv_ref[...]


def _proj_call(x, wqt, bq2, wkvt, bkv2):
    return pl.pallas_call(
        _proj_body,
        grid=(N // BN,),
        in_specs=[
            pl.BlockSpec((BN, D), lambda i: (i, 0)),
            pl.BlockSpec((D, H * DK), lambda i: (0, 0)),
            pl.BlockSpec((1, H * DK), lambda i: (0, 0)),
            pl.BlockSpec((D, 2 * H * DK), lambda i: (0, 0)),
            pl.BlockSpec((1, 2 * H * DK), lambda i: (0, 0)),
        ],
        out_specs=[
            pl.BlockSpec((BN, H * DK), lambda i: (i, 0)),
            pl.BlockSpec((BN, 2 * H * DK), lambda i: (i, 0)),
        ],
        out_shape=[
            jax.ShapeDtypeStruct((N, H * DK), jnp.float32),
            jax.ShapeDtypeStruct((N, 2 * H * DK), jnp.float32),
        ],
    )(x, wqt, bq2, wkvt, bkv2)


def _final_body(slab_ref, w_ref, b_ref, o_ref):
    acc = jnp.zeros((BN, D), jnp.float32)
    for h in range(H):
        blk = slab_ref[h]
        msgs = blk[:, 0:DV]
        s = blk[:, DV:DV + 1]
        rec = jnp.where(s > 0, 1.0 / s, 0.0)
        mn = msgs * rec
        acc = acc + jnp.dot(mn, w_ref[h * DV:(h + 1) * DV, :],
                            precision=lax.Precision.HIGHEST,
                            preferred_element_type=jnp.float32)
    o_ref[...] = acc + b_ref[...]


def _final_call(slab, wffp, bff2):
    return pl.pallas_call(
        _final_body,
        grid=(N // BN,),
        in_specs=[
            pl.BlockSpec((H, BN, SLAB_W), lambda i: (0, i, 0)),
            pl.BlockSpec((H * DV, D), lambda i: (0, 0)),
            pl.BlockSpec((1, D), lambda i: (0, 0)),
        ],
        out_specs=pl.BlockSpec((BN, D), lambda i: (i, 0)),
        out_shape=jax.ShapeDtypeStruct((N, D), jnp.float32),
    )(slab, wffp, bff2)


def _sc_body(recv_h, send_h, q8_h, kv8_h, out_h,
             idx_r0, idx_r1, idx_s0, idx_s1,
             qidx0, qidx1, kvidx0, kvidx1, scat0, scat1,
             qg0, qg1, kvg0, kvg1, msg0, msg1, zbuf, slab,
             sidx0, sidx1, sg0, sg1, ssc0, ssc1):
    c_ax = lax.axis_index("c")
    s_ax = lax.axis_index("s")
    idx_r = (idx_r0, idx_r1)
    idx_s = (idx_s0, idx_s1)
    qidx = (qidx0, qidx1)
    kvidx = (kvidx0, kvidx1)
    scat = (scat0, scat1)
    qg = (qg0, qg1)
    kvg = (kvg0, kvg1)
    msg = (msg0, msg1)
    sidx = (sidx0, sidx1)
    sg = (sg0, sg1)
    ssc = (ssc0, ssc1)

    tile_base = s_ax * ET
    row0 = s_ax * NR

    def zb_row(i, carry):
        for k in range(SLAB_W // 16):
            zbuf[i, pl.ds(16 * k, 16)] = jnp.zeros((16,), jnp.float32)
        return carry

    lax.fori_loop(0, 25, zb_row, 0)

    def load_idx(chunk, slot):
        base = tile_base + chunk * B
        pltpu.async_copy(recv_h.at[pl.ds(base, B)], idx_r[slot], sidx[slot])
        pltpu.async_copy(send_h.at[pl.ds(base, B)], idx_s[slot], sidx[slot])

    def wait_idx(slot):
        pltpu.make_async_copy(recv_h.at[pl.ds(0, B)], idx_r[slot],
                              sidx[slot]).wait()
        pltpu.make_async_copy(send_h.at[pl.ds(0, B)], idx_s[slot],
                              sidx[slot]).wait()

    def issue_gathers(g_dyn, slot):
        wait_idx(slot)
        for v in range(B // 16):
            sl = pl.ds(16 * v, 16)
            qidx[slot][sl] = idx_r[slot][sl] * H + g_dyn
            kvidx[slot][sl] = idx_s[slot][sl] * H + g_dyn
        pltpu.async_copy(q8_h.at[qidx[slot]], qg[slot], sg[slot])
        pltpu.async_copy(kv8_h.at[kvidx[slot]], kvg[slot], sg[slot])

    def wait_gathers(slot):
        pltpu.make_async_copy(q8_h.at[qidx[slot]], qg[slot], sg[slot]).wait()
        pltpu.make_async_copy(kv8_h.at[kvidx[slot]], kvg[slot],
                              sg[slot]).wait()

    def wait_scatter(slot):
        pltpu.make_async_copy(msg[slot], slab.at[scat[slot]],
                              ssc[slot]).wait()

    def compute(slot):
        qgr, kvr, msr = qg[slot], kvg[slot], msg[slot]
        for v in range(B // 16):
            sl = pl.ds(16 * v, 16)
            scat[slot][sl] = idx_r[slot][sl]

        def edge(e, carry):
            acc = qgr[e, pl.ds(0, 16)] * kvr[e, pl.ds(0, 16)]
            for k in range(1, 4):
                acc = acc + (qgr[e, pl.ds(16 * k, 16)] *
                             kvr[e, pl.ds(16 * k, 16)])
            t = jnp.sum(acc) * jnp.float32(0.125)
            att_v = jnp.exp(jnp.full((16,), t, jnp.float32))
            for k in range(4):
                msr[e, pl.ds(16 * k, 16)] = (
                    att_v * kvr[e, pl.ds(64 + 16 * k, 16)])
            lane = lax.iota(jnp.int32, 16)
            pair = jnp.where(lane == 0, att_v, jnp.float32(0.))
            msr[e, pl.ds(64, 16)] = pair
            return carry

        lax.fori_loop(0, B, edge, 0)
        pltpu.async_copy(msg[slot], slab.at[scat[slot]], ssc[slot], add=True)

    def run_pass(p, carry):
        g_dyn = c_ax * PP + p
        for j in range(NR // 25):
            pltpu.sync_copy(zbuf, slab.at[pl.ds(row0 + j * 25, 25)])
        plsc.subcore_barrier()

        load_idx(jnp.int32(0), 0)
        load_idx(jnp.int32(1), 1)
        issue_gathers(g_dyn, 0)

        def substep(c, slot, oslot):
            @pl.when(c <= NCH - 2)
            def _():
                issue_gathers(g_dyn, oslot)

            wait_gathers(slot)

            @pl.when(c >= 2)
            def _():
                wait_scatter(slot)

            compute(slot)

            @pl.when(c <= NCH - 3)
            def _():
                load_idx(c + 2, slot)

        def pair_body(i, carry2):
            c0 = i * 2
            substep(c0, 0, 1)
            substep(c0 + 1, 1, 0)
            return carry2

        lax.fori_loop(0, NCH // 2, pair_body, 0)
        wait_scatter(0)
        wait_scatter(1)
        plsc.subcore_barrier()
        for j in range(NR // ZB):
            rows = pl.ds(row0 + j * ZB, ZB)
            pltpu.sync_copy(slab.at[rows], out_h.at[g_dyn, rows])
        plsc.subcore_barrier()
        return carry

    lax.fori_loop(0, PP, run_pass, 0)


def _sc_call(receivers, senders, q8r, kv8r):
    mesh = plsc.VectorSubcoreMesh(core_axis_name="c", subcore_axis_name="s",
                                  num_cores=NC, num_subcores=NS)
    f = pl.kernel(
        _sc_body,
        out_type=jax.ShapeDtypeStruct((H, N, SLAB_W), jnp.float32),
        mesh=mesh,
        scratch_types=[
            pltpu.VMEM((B,), jnp.int32), pltpu.VMEM((B,), jnp.int32),
            pltpu.VMEM((B,), jnp.int32), pltpu.VMEM((B,), jnp.int32),
            pltpu.VMEM((B,), jnp.int32), pltpu.VMEM((B,), jnp.int32),
            pltpu.VMEM((B,), jnp.int32), pltpu.VMEM((B,), jnp.int32),
            pltpu.VMEM((B,), jnp.int32), pltpu.VMEM((B,), jnp.int32),
            pltpu.VMEM((B, DK), jnp.float32),
            pltpu.VMEM((B, DK), jnp.float32),
            pltpu.VMEM((B, 2 * DK), jnp.float32),
            pltpu.VMEM((B, 2 * DK), jnp.float32),
            pltpu.VMEM((B, SLAB_W), jnp.float32),
            pltpu.VMEM((B, SLAB_W), jnp.float32),
            pltpu.VMEM((25, SLAB_W), jnp.float32),
            pltpu.VMEM_SHARED((N, SLAB_W), jnp.float32),
            pltpu.SemaphoreType.DMA, pltpu.SemaphoreType.DMA,
            pltpu.SemaphoreType.DMA, pltpu.SemaphoreType.DMA,
            pltpu.SemaphoreType.DMA, pltpu.SemaphoreType.DMA,
        ],
        compiler_params=pltpu.CompilerParams(use_tc_tiling_on_sc=False,
                                             needs_layout_passes=False),
    )
    return f(receivers, senders, q8r, kv8r)


def kernel(x, edge_index, Wk, bk, Wq, bq, Wv, bv, Wff, bff):
    x = x.astype(jnp.float32)
    ei = edge_index.astype(jnp.int32)
    senders = ei[0]
    receivers = ei[1]

    wqt = Wq.T
    bq2 = bq.reshape(1, H * DK)
    wkv = jnp.stack([Wk.reshape(H, DK, D), Wv.reshape(H, DV, D)],
                    axis=1).reshape(2 * H * DK, D)
    bkv2 = jnp.stack([bk.reshape(H, DK), bv.reshape(H, DV)],
                     axis=1).reshape(1, 2 * H * DK)

    q8, kv8 = _proj_call(x, wqt, bq2, wkv.T, bkv2)
    q8r = q8.reshape(H * N, DK)
    kv8r = kv8.reshape(H * N, 2 * DK)

    slab = _sc_call(receivers, senders, q8r, kv8r)
    out = _final_call(slab, Wff.T, bff.reshape(1, D))
    return out
